# Initial kernel scaffold; baseline (speedup 1.0000x reference)
import jax
import jax.numpy as jnp
from jax.experimental import pallas as pl


def kernel(index, length, data):
    raise NotImplementedError("write your pallas kernel here")


# speedup vs baseline: 4.6593x; 4.6593x over previous
import functools

import jax
import jax.numpy as jnp
from jax import lax
from jax.experimental import pallas as pl
from jax.experimental.pallas import tpu as pltpu
from jax.experimental.pallas import tpu_sc as plsc

CYCLE_LEN = 1440
CHANNEL = 64
BATCH = 1024
SEQ_LEN = 336

_NUM_CORES = 2
_NUM_SUBCORES = 16
_NUM_WORKERS = _NUM_CORES * _NUM_SUBCORES
_B_PER_W = BATCH // _NUM_WORKERS
_EXT_LEN = CYCLE_LEN + SEQ_LEN
_ROW = CHANNEL
_OUT_PER_B = SEQ_LEN * CHANNEL


@functools.partial(
    pl.kernel,
    mesh=plsc.VectorSubcoreMesh(core_axis_name="c", subcore_axis_name="s"),
    out_type=jax.ShapeDtypeStruct((BATCH * SEQ_LEN * CHANNEL,), jnp.float32),
    scratch_types=[
        pltpu.VMEM((_EXT_LEN * _ROW,), jnp.float32),
        pltpu.VMEM((_B_PER_W,), jnp.int32),
        pltpu.SemaphoreType.DMA,
    ],
)
def _cycle_gather(index_hbm, data_hbm, out_hbm, ext_v, idx_v, sem):
    wid = lax.axis_index("s") * _NUM_CORES + lax.axis_index("c")
    base = wid * _B_PER_W

    pltpu.sync_copy(data_hbm, ext_v.at[pl.ds(0, CYCLE_LEN * _ROW)])
    pltpu.sync_copy(data_hbm.at[pl.ds(0, SEQ_LEN * _ROW)],
                    ext_v.at[pl.ds(CYCLE_LEN * _ROW, SEQ_LEN * _ROW)])
    pltpu.sync_copy(index_hbm.at[pl.ds(base, _B_PER_W)], idx_v)

    copies = []
    for chunk in range(_B_PER_W // 16):
        idx_vec = idx_v[pl.ds(chunk * 16, 16)]
        for j in range(16):
            i = chunk * 16 + j
            idx = idx_vec[j]
            copies.append(
                pltpu.async_copy(
                    ext_v.at[pl.ds(idx * _ROW, _OUT_PER_B)],
                    out_hbm.at[pl.ds((base + i) * _OUT_PER_B, _OUT_PER_B)],
                    sem))
    for c in copies:
        c.wait()


def kernel(index, length, data):
    del length
    out_flat = _cycle_gather(index, data.reshape(-1))
    return out_flat.reshape(BATCH, SEQ_LEN, CHANNEL)
